# Initial kernel scaffold; baseline (speedup 1.0000x reference)
import jax
import jax.numpy as jnp
from jax.experimental import pallas as pl


def kernel(feat, state, edge_index, W_zr_nbr, W_zr_self, b_zr, W_c_nbr, W_c_self, b_c):
    raise NotImplementedError("write your pallas kernel here")


# speedup vs baseline: 6.3978x; 6.3978x over previous
import functools

import jax
import jax.numpy as jnp
from jax import lax
from jax.experimental import pallas as pl
from jax.experimental.pallas import tpu as pltpu
from jax.experimental.pallas import tpu_sc as plsc

N = 10000
E = 320000
D = 128

NC = 2
NS = 16
NW = NC * NS
EPW = E // NW
K = 200
NCHUNK = EPW // K
RPT = 624
TAIL = N - NS * RPT


def _make_agg(n_tables):
  mesh = plsc.VectorSubcoreMesh(core_axis_name="c", subcore_axis_name="s")
  out_type = tuple(
      jax.ShapeDtypeStruct((NC * N, D), jnp.float32) for _ in range(n_tables)
  )
  scratch = [
      pltpu.VMEM((K,), jnp.int32),
      pltpu.VMEM((K,), jnp.int32),
      pltpu.VMEM((K, D), jnp.float32),
      pltpu.VMEM_SHARED((N, D), jnp.float32),
      pltpu.SemaphoreType.DMA,
  ]

  @functools.partial(pl.kernel, out_type=out_type, mesh=mesh,
                     scratch_types=scratch)
  def agg(src_hbm, dst_hbm, *rest):
    tables = rest[:n_tables]
    zeros_hbm = rest[n_tables]
    outs = rest[n_tables + 1: 1 + 2 * n_tables]
    idx_s, idx_d, rows, acc, sem = rest[1 + 2 * n_tables:]

    c = lax.axis_index("c")
    s = lax.axis_index("s")
    ebase = (c * NS + s) * EPW
    row0 = s * RPT

    for t in range(n_tables):
      pltpu.sync_copy(zeros_hbm, acc.at[pl.ds(row0, RPT)])

      @pl.when(s == NS - 1)
      def _zero_tail():
        pltpu.sync_copy(zeros_hbm.at[pl.ds(0, TAIL)],
                        acc.at[pl.ds(NS * RPT, TAIL)])

      plsc.subcore_barrier()

      def body(i, carry):
        e0 = ebase + i * K
        pltpu.sync_copy(src_hbm.at[pl.ds(e0, K)], idx_s)
        pltpu.sync_copy(dst_hbm.at[pl.ds(e0, K)], idx_d)
        pltpu.async_copy(tables[t].at[idx_s], rows, sem).wait()
        pltpu.sync_copy(rows, acc.at[idx_d], add=True)
        return carry

      lax.fori_loop(0, NCHUNK, body, 0)
      plsc.subcore_barrier()
      pltpu.sync_copy(acc.at[pl.ds(row0, RPT)],
                      outs[t].at[pl.ds(c * N + row0, RPT)])

      @pl.when(s == NS - 1)
      def _dump_tail():
        pltpu.sync_copy(acc.at[pl.ds(NS * RPT, TAIL)],
                        outs[t].at[pl.ds(c * N + NS * RPT, TAIL)])

  return agg


_agg2 = _make_agg(2)
_agg1 = _make_agg(1)

_R = 1000


def _gate1_body(p0, p1, q0, q1, f, st, wnt, wnb, wst, wsb, b,
                z_o, rs_o, ps_o):
  ps = p0[...] + p1[...]
  qs = q0[...] + q1[...]
  acc = jnp.dot(ps, wnt[...], preferred_element_type=jnp.float32)
  acc += jnp.dot(qs, wnb[...], preferred_element_type=jnp.float32)
  acc += jnp.dot(f[...], wst[...], preferred_element_type=jnp.float32)
  acc += jnp.dot(st[...], wsb[...], preferred_element_type=jnp.float32)
  zr = jax.nn.sigmoid(acc + b[...])
  z_o[...] = zr[:, :D]
  rs_o[...] = zr[:, D:] * st[...]
  ps_o[...] = ps


def _gate2_body(ps, s0, s1, f, rs, z, st, wnt, wnb, wst, wsb, b, out):
  acc = jnp.dot(ps[...], wnt[...], preferred_element_type=jnp.float32)
  acc += jnp.dot(s0[...] + s1[...], wnb[...], preferred_element_type=jnp.float32)
  acc += jnp.dot(f[...], wst[...], preferred_element_type=jnp.float32)
  acc += jnp.dot(rs[...], wsb[...], preferred_element_type=jnp.float32)
  c = jnp.tanh(acc + b[...])
  zz = z[...]
  out[...] = zz * st[...] + (1.0 - zz) * c


def _row_spec(off_blocks):
  return pl.BlockSpec((_R, D), lambda i, o=off_blocks: (o + i, 0))


def _full_spec(shape):
  return pl.BlockSpec(shape, lambda i: (0, 0))


def _gate1(p2, q2, feat, state, wnt, wnb, wst, wsb, b):
  nb = N // _R
  grid = (nb,)
  in_specs = [
      _row_spec(0), _row_spec(nb),
      _row_spec(0), _row_spec(nb),
      _row_spec(0), _row_spec(0),
      _full_spec((D, 2 * D)), _full_spec((D, 2 * D)),
      _full_spec((D, 2 * D)), _full_spec((D, 2 * D)),
      _full_spec((1, 2 * D)),
  ]
  out_specs = [_row_spec(0), _row_spec(0), _row_spec(0)]
  out_shape = [jax.ShapeDtypeStruct((N, D), jnp.float32)] * 3
  return pl.pallas_call(
      _gate1_body, grid=grid, in_specs=in_specs, out_specs=out_specs,
      out_shape=out_shape,
  )(p2, p2, q2, q2, feat, state, wnt, wnb, wst, wsb, b)


def _gate2(ps, s2, feat, rs, z, state, wnt, wnb, wst, wsb, b):
  nb = N // _R
  grid = (nb,)
  in_specs = [
      _row_spec(0),
      _row_spec(0), _row_spec(nb),
      _row_spec(0), _row_spec(0), _row_spec(0), _row_spec(0),
      _full_spec((D, D)), _full_spec((D, D)),
      _full_spec((D, D)), _full_spec((D, D)),
      _full_spec((1, D)),
  ]
  return pl.pallas_call(
      _gate2_body, grid=grid, in_specs=in_specs,
      out_specs=_row_spec(0),
      out_shape=jax.ShapeDtypeStruct((N, D), jnp.float32),
  )(ps, s2, s2, feat, rs, z, state, wnt, wnb, wst, wsb, b)


def kernel(feat, state, edge_index, W_zr_nbr, W_zr_self, b_zr,
           W_c_nbr, W_c_self, b_c):
  src = edge_index[0].astype(jnp.int32)
  dst = edge_index[1].astype(jnp.int32)
  zeros = jnp.zeros((RPT, D), jnp.float32)

  p2, q2 = _agg2(src, dst, feat, state, zeros)
  z, rs, ps = _gate1(
      p2, q2, feat, state,
      W_zr_nbr[:D], W_zr_nbr[D:], W_zr_self[:D], W_zr_self[D:],
      b_zr.reshape(1, 2 * D),
  )
  (s2,) = _agg1(src, dst, rs, zeros)
  return _gate2(
      ps, s2, feat, rs, z, state,
      W_c_nbr[:D], W_c_nbr[D:], W_c_self[:D], W_c_self[D:],
      b_c.reshape(1, D),
  )


# speedup vs baseline: 10.0808x; 1.5757x over previous
import functools

import jax
import jax.numpy as jnp
from jax import lax
from jax.experimental import pallas as pl
from jax.experimental.pallas import tpu as pltpu
from jax.experimental.pallas import tpu_sc as plsc

N = 10000
E = 320000
D = 128

NC = 2
NS = 16
NW = NC * NS
EPW = E // NW
K = 80
NCHUNK = EPW // K
RPT = 624
TAIL = N - NS * RPT


def _make_agg(n_tables):
  mesh = plsc.VectorSubcoreMesh(core_axis_name="c", subcore_axis_name="s")
  out_type = tuple(
      jax.ShapeDtypeStruct((NC * N, D), jnp.float32) for _ in range(n_tables)
  )
  scratch = [
      pltpu.VMEM((EPW,), jnp.int32),
      pltpu.VMEM((K,), jnp.int32),
      pltpu.VMEM((K,), jnp.int32),
      pltpu.VMEM((K, D), jnp.float32),
      pltpu.VMEM((K, D), jnp.float32),
      pltpu.VMEM_SHARED((N, D), jnp.float32),
      pltpu.SemaphoreType.DMA,
      pltpu.SemaphoreType.DMA,
  ]

  @functools.partial(pl.kernel, out_type=out_type, mesh=mesh,
                     scratch_types=scratch)
  def agg(src_hbm, dst_hbm, *rest):
    tables = rest[:n_tables]
    zeros_hbm = rest[n_tables]
    outs = rest[n_tables + 1: 1 + 2 * n_tables]
    idx_s, idx_da, idx_db, rows_a, rows_b, acc, sem_a, sem_b = (
        rest[1 + 2 * n_tables:])

    c = lax.axis_index("c")
    s = lax.axis_index("s")
    ebase = (c * NS + s) * EPW
    row0 = s * RPT

    pltpu.sync_copy(src_hbm.at[pl.ds(ebase, EPW)], idx_s)

    def issue(i, rows_buf, idx_buf, sem, table):
      pltpu.async_copy(table.at[idx_s.at[pl.ds(i * K, K)]], rows_buf, sem)
      pltpu.async_copy(dst_hbm.at[pl.ds(ebase + i * K, K)], idx_buf, sem)

    def drain_scatter(rows_buf, idx_buf, sem, table):
      pltpu.make_async_copy(table.at[idx_s.at[pl.ds(0, K)]],
                            rows_buf, sem).wait()
      pltpu.make_async_copy(dst_hbm.at[pl.ds(0, K)], idx_buf, sem).wait()
      pltpu.sync_copy(rows_buf, acc.at[idx_buf], add=True)

    for t in range(n_tables):
      pltpu.sync_copy(zeros_hbm, acc.at[pl.ds(row0, RPT)])

      @pl.when(s == NS - 1)
      def _zero_tail():
        pltpu.sync_copy(zeros_hbm.at[pl.ds(0, TAIL)],
                        acc.at[pl.ds(NS * RPT, TAIL)])

      plsc.subcore_barrier()

      table = tables[t]
      issue(0, rows_a, idx_da, sem_a, table)

      def body(j, carry):
        issue(2 * j + 1, rows_b, idx_db, sem_b, table)
        drain_scatter(rows_a, idx_da, sem_a, table)
        issue(2 * j + 2, rows_a, idx_da, sem_a, table)
        drain_scatter(rows_b, idx_db, sem_b, table)
        return carry

      lax.fori_loop(0, (NCHUNK - 1) // 2, body, 0)
      drain_scatter(rows_a, idx_da, sem_a, table)
      plsc.subcore_barrier()
      pltpu.sync_copy(acc.at[pl.ds(row0, RPT)],
                      outs[t].at[pl.ds(c * N + row0, RPT)])

      @pl.when(s == NS - 1)
      def _dump_tail():
        pltpu.sync_copy(acc.at[pl.ds(NS * RPT, TAIL)],
                        outs[t].at[pl.ds(c * N + NS * RPT, TAIL)])

  return agg


_agg2 = _make_agg(2)
_agg1 = _make_agg(1)

_R = 1000


def _gate1_body(p0, p1, q0, q1, f, st, wnt, wnb, wst, wsb, b,
                z_o, rs_o, ps_o):
  ps = p0[...] + p1[...]
  qs = q0[...] + q1[...]
  acc = jnp.dot(ps, wnt[...], preferred_element_type=jnp.float32)
  acc += jnp.dot(qs, wnb[...], preferred_element_type=jnp.float32)
  acc += jnp.dot(f[...], wst[...], preferred_element_type=jnp.float32)
  acc += jnp.dot(st[...], wsb[...], preferred_element_type=jnp.float32)
  zr = jax.nn.sigmoid(acc + b[...])
  z_o[...] = zr[:, :D]
  rs_o[...] = zr[:, D:] * st[...]
  ps_o[...] = ps


def _gate2_body(ps, s0, s1, f, rs, z, st, wnt, wnb, wst, wsb, b, out):
  acc = jnp.dot(ps[...], wnt[...], preferred_element_type=jnp.float32)
  acc += jnp.dot(s0[...] + s1[...], wnb[...], preferred_element_type=jnp.float32)
  acc += jnp.dot(f[...], wst[...], preferred_element_type=jnp.float32)
  acc += jnp.dot(rs[...], wsb[...], preferred_element_type=jnp.float32)
  c = jnp.tanh(acc + b[...])
  zz = z[...]
  out[...] = zz * st[...] + (1.0 - zz) * c


def _row_spec(off_blocks):
  return pl.BlockSpec((_R, D), lambda i, o=off_blocks: (o + i, 0))


def _full_spec(shape):
  return pl.BlockSpec(shape, lambda i: (0, 0))


def _gate1(p2, q2, feat, state, wnt, wnb, wst, wsb, b):
  nb = N // _R
  grid = (nb,)
  in_specs = [
      _row_spec(0), _row_spec(nb),
      _row_spec(0), _row_spec(nb),
      _row_spec(0), _row_spec(0),
      _full_spec((D, 2 * D)), _full_spec((D, 2 * D)),
      _full_spec((D, 2 * D)), _full_spec((D, 2 * D)),
      _full_spec((1, 2 * D)),
  ]
  out_specs = [_row_spec(0), _row_spec(0), _row_spec(0)]
  out_shape = [jax.ShapeDtypeStruct((N, D), jnp.float32)] * 3
  return pl.pallas_call(
      _gate1_body, grid=grid, in_specs=in_specs, out_specs=out_specs,
      out_shape=out_shape,
  )(p2, p2, q2, q2, feat, state, wnt, wnb, wst, wsb, b)


def _gate2(ps, s2, feat, rs, z, state, wnt, wnb, wst, wsb, b):
  nb = N // _R
  grid = (nb,)
  in_specs = [
      _row_spec(0),
      _row_spec(0), _row_spec(nb),
      _row_spec(0), _row_spec(0), _row_spec(0), _row_spec(0),
      _full_spec((D, D)), _full_spec((D, D)),
      _full_spec((D, D)), _full_spec((D, D)),
      _full_spec((1, D)),
  ]
  return pl.pallas_call(
      _gate2_body, grid=grid, in_specs=in_specs,
      out_specs=_row_spec(0),
      out_shape=jax.ShapeDtypeStruct((N, D), jnp.float32),
  )(ps, s2, s2, feat, rs, z, state, wnt, wnb, wst, wsb, b)


def kernel(feat, state, edge_index, W_zr_nbr, W_zr_self, b_zr,
           W_c_nbr, W_c_self, b_c):
  src = edge_index[0].astype(jnp.int32)
  dst = edge_index[1].astype(jnp.int32)
  zeros = jnp.zeros((RPT, D), jnp.float32)

  p2, q2 = _agg2(src, dst, feat, state, zeros)
  z, rs, ps = _gate1(
      p2, q2, feat, state,
      W_zr_nbr[:D], W_zr_nbr[D:], W_zr_self[:D], W_zr_self[D:],
      b_zr.reshape(1, 2 * D),
  )
  (s2,) = _agg1(src, dst, rs, zeros)
  return _gate2(
      ps, s2, feat, rs, z, state,
      W_c_nbr[:D], W_c_nbr[D:], W_c_self[:D], W_c_self[D:],
      b_c.reshape(1, D),
  )
